# Initial kernel scaffold; baseline (speedup 1.0000x reference)
import jax
import jax.numpy as jnp
from jax.experimental import pallas as pl


def kernel(locations, pred_cls, pred_boxes, pred_centerness, image_sizes):
    raise NotImplementedError("write your pallas kernel here")


# speedup vs baseline: 7.9579x; 7.9579x over previous
import jax
import jax.numpy as jnp
from jax import lax
from jax.experimental import pallas as pl
from jax.experimental.pallas import tpu as pltpu

_NMS_THRESH = 0.6
_CONF = 0.05
_TOPN = 1000
_KP = 1024
_RCHUNK = 256


def _decode_rows(lx, ly, bl, bt, br, bb, lab, wv, hv):
    x1 = jnp.clip(lx - bl, 0.0, wv)
    y1 = jnp.clip(ly - bt, 0.0, hv)
    x2 = jnp.clip(lx + br, 0.0, wv)
    y2 = jnp.clip(ly + bb, 0.0, hv)
    off = lab * (jnp.maximum(wv, hv) + 1.0)
    return x1, y1, x2, y2, off


def _nms_body(cand_ref, candt_ref, out_ref, a_ref):
    c = cand_ref[0]
    x1, y1, x2, y2, off = _decode_rows(
        c[0:1, :], c[1:2, :], c[2:3, :], c[3:4, :], c[4:5, :], c[5:6, :],
        c[6:7, :], c[7:8, :], c[8:9, :])
    out_ref[0, 0:1, :] = x1
    out_ref[0, 1:2, :] = y1
    out_ref[0, 2:3, :] = x2
    out_ref[0, 3:4, :] = y2
    nx1, ny1, nx2, ny2 = x1 + off, y1 + off, x2 + off, y2 + off
    area_r = jnp.maximum(nx2 - nx1, 0.0) * jnp.maximum(ny2 - ny1, 0.0)

    ct = candt_ref[0]
    x1c, y1c, x2c, y2c, offc = _decode_rows(
        ct[:, 0:1], ct[:, 1:2], ct[:, 2:3], ct[:, 3:4], ct[:, 4:5],
        ct[:, 5:6], ct[:, 6:7], ct[:, 7:8], ct[:, 8:9])
    nx1c, ny1c, nx2c, ny2c = x1c + offc, y1c + offc, x2c + offc, y2c + offc
    area_c = jnp.maximum(nx2c - nx1c, 0.0) * jnp.maximum(ny2c - ny1c, 0.0)

    for r0 in range(0, _KP, _RCHUNK):
        r1 = r0 + _RCHUNK
        ix1 = jnp.maximum(nx1c[r0:r1, :], nx1)
        iy1 = jnp.maximum(ny1c[r0:r1, :], ny1)
        ix2 = jnp.minimum(nx2c[r0:r1, :], nx2)
        iy2 = jnp.minimum(ny2c[r0:r1, :], ny2)
        inter = jnp.maximum(ix2 - ix1, 0.0) * jnp.maximum(iy2 - iy1, 0.0)
        union = area_c[r0:r1, :] + area_r - inter
        iou = inter / (union + 1e-9)
        rowi = lax.broadcasted_iota(jnp.int32, (_RCHUNK, _KP), 0) + r0
        coli = lax.broadcasted_iota(jnp.int32, (_RCHUNK, _KP), 1)
        a_ref[r0:r1, :] = jnp.where(
            (iou > _NMS_THRESH) & (coli < rowi), 1.0, 0.0)

    iot = lax.broadcasted_iota(jnp.int32, (1, _KP), 1)

    def body(i, keep):
        arow = a_ref[pl.ds(i, 1), :]
        sup = jnp.sum(arow * keep)
        kv = jnp.where(sup > 0.0, 0.0, 1.0)
        return jnp.where(iot == i, kv, keep)

    keep = lax.fori_loop(0, _TOPN, body, jnp.ones((1, _KP), jnp.float32))
    out_ref[0, 4:5, :] = keep
    out_ref[0, 5:8, :] = jnp.zeros((3, _KP), jnp.float32)


def _run_nms(cand, candt, B):
    return pl.pallas_call(
        _nms_body,
        grid=(B,),
        in_specs=[
            pl.BlockSpec((1, 16, _KP), lambda b: (b, 0, 0)),
            pl.BlockSpec((1, _KP, 16), lambda b: (b, 0, 0)),
        ],
        out_specs=pl.BlockSpec((1, 8, _KP), lambda b: (b, 0, 0)),
        out_shape=jax.ShapeDtypeStruct((B, 8, _KP), jnp.float32),
        scratch_shapes=[pltpu.VMEM((_KP, _KP), jnp.float32)],
    )(cand, candt)


def kernel(locations, pred_cls, pred_boxes, pred_centerness, image_sizes):
    B, H, W, C = pred_cls.shape
    N = H * W
    K = _TOPN
    cls = jax.nn.sigmoid(pred_cls.reshape(B, N, C))
    ctr = jax.nn.sigmoid(pred_centerness.reshape(B, N))
    scores = cls * ctr[:, :, None]
    scores = scores * (cls > _CONF).astype(scores.dtype)
    vals, idx = lax.top_k(scores.reshape(B, N * C), K)
    loc_idx = idx // C
    labels = (idx % C) + 1
    labf = labels.astype(jnp.float32)
    pb = jnp.take_along_axis(
        pred_boxes.reshape(B, N, 4), loc_idx[..., None], axis=1)
    ploc = locations[loc_idx]
    hf = image_sizes[0, 0].astype(jnp.float32)
    wf = image_sizes[0, 1].astype(jnp.float32)

    pad = _KP - K
    rows = [ploc[..., 0], ploc[..., 1], pb[..., 0], pb[..., 1], pb[..., 2],
            pb[..., 3], labf]
    rows = [jnp.pad(r, ((0, 0), (0, pad)))[:, None, :] for r in rows]
    rows.append(jnp.broadcast_to(wf, (B, 1, _KP)))
    rows.append(jnp.broadcast_to(hf, (B, 1, _KP)))
    rows.append(jnp.zeros((B, 7, _KP), jnp.float32))
    cand = jnp.concatenate(rows, axis=1)
    candt = jnp.swapaxes(cand, 1, 2)

    out = _run_nms(cand, candt, B)
    det = jnp.swapaxes(out[:, 0:4, :K], 1, 2)
    keep = out[:, 4, :K]

    fsc = vals * keep
    fvals, fidx = lax.top_k(fsc, K)
    fdet = jnp.take_along_axis(det, fidx[..., None], axis=1)
    flab = jnp.take_along_axis(labf, fidx, axis=1)
    flab = flab * (fvals > 0).astype(jnp.float32)
    return jnp.concatenate([fdet, fvals[..., None], flab[..., None]], axis=2)


# speedup vs baseline: 13.4530x; 1.6905x over previous
import jax
import jax.numpy as jnp
from jax import lax
from jax.experimental import pallas as pl
from jax.experimental.pallas import tpu as pltpu

_NMS_THRESH = 0.6
_CONF = 0.05
_TOPN = 1000
_KP = 1024
_RCHUNK = 256


def _decode_rows(lx, ly, bl, bt, br, bb, lab, wv, hv):
    x1 = jnp.clip(lx - bl, 0.0, wv)
    y1 = jnp.clip(ly - bt, 0.0, hv)
    x2 = jnp.clip(lx + br, 0.0, wv)
    y2 = jnp.clip(ly + bb, 0.0, hv)
    off = lab * (jnp.maximum(wv, hv) + 1.0)
    return x1, y1, x2, y2, off


def _locmax_body(cls_ref, ctr_ref, out_ref):
    s = jax.nn.sigmoid(cls_ref[0])
    sc = s * jax.nn.sigmoid(ctr_ref[0])
    sc = sc * (s > _CONF).astype(sc.dtype)
    out_ref[0] = jnp.max(sc, axis=1, keepdims=True)


def _run_locmax(cls3, ctr3, B, N, C):
    return pl.pallas_call(
        _locmax_body,
        grid=(B,),
        in_specs=[
            pl.BlockSpec((1, N, C), lambda b: (b, 0, 0)),
            pl.BlockSpec((1, N, 1), lambda b: (b, 0, 0)),
        ],
        out_specs=pl.BlockSpec((1, N, 1), lambda b: (b, 0, 0)),
        out_shape=jax.ShapeDtypeStruct((B, N, 1), jnp.float32),
    )(cls3, ctr3)


def _nms_body(cand_ref, candt_ref, out_ref, a_ref):
    c = cand_ref[0]
    x1, y1, x2, y2, off = _decode_rows(
        c[0:1, :], c[1:2, :], c[2:3, :], c[3:4, :], c[4:5, :], c[5:6, :],
        c[6:7, :], c[7:8, :], c[8:9, :])
    out_ref[0, 0:1, :] = x1
    out_ref[0, 1:2, :] = y1
    out_ref[0, 2:3, :] = x2
    out_ref[0, 3:4, :] = y2
    nx1, ny1, nx2, ny2 = x1 + off, y1 + off, x2 + off, y2 + off
    area_r = jnp.maximum(nx2 - nx1, 0.0) * jnp.maximum(ny2 - ny1, 0.0)

    ct = candt_ref[0]
    x1c, y1c, x2c, y2c, offc = _decode_rows(
        ct[:, 0:1], ct[:, 1:2], ct[:, 2:3], ct[:, 3:4], ct[:, 4:5],
        ct[:, 5:6], ct[:, 6:7], ct[:, 7:8], ct[:, 8:9])
    nx1c, ny1c, nx2c, ny2c = x1c + offc, y1c + offc, x2c + offc, y2c + offc
    area_c = jnp.maximum(nx2c - nx1c, 0.0) * jnp.maximum(ny2c - ny1c, 0.0)

    for r0 in range(0, _KP, _RCHUNK):
        r1 = r0 + _RCHUNK
        ix1 = jnp.maximum(nx1c[r0:r1, :], nx1)
        iy1 = jnp.maximum(ny1c[r0:r1, :], ny1)
        ix2 = jnp.minimum(nx2c[r0:r1, :], nx2)
        iy2 = jnp.minimum(ny2c[r0:r1, :], ny2)
        inter = jnp.maximum(ix2 - ix1, 0.0) * jnp.maximum(iy2 - iy1, 0.0)
        union = area_c[r0:r1, :] + area_r - inter
        iou = inter / (union + 1e-9)
        rowi = lax.broadcasted_iota(jnp.int32, (_RCHUNK, _KP), 0) + r0
        coli = lax.broadcasted_iota(jnp.int32, (_RCHUNK, _KP), 1)
        a_ref[r0:r1, :] = jnp.where(
            (iou > _NMS_THRESH) & (coli < rowi), 1.0, 0.0)

    iot = lax.broadcasted_iota(jnp.int32, (1, _KP), 1)

    def body(i, keep):
        arow = a_ref[pl.ds(i, 1), :]
        sup = jnp.sum(arow * keep)
        kv = jnp.where(sup > 0.0, 0.0, 1.0)
        return jnp.where(iot == i, kv, keep)

    keep = lax.fori_loop(0, _TOPN, body, jnp.ones((1, _KP), jnp.float32))
    out_ref[0, 4:5, :] = keep
    out_ref[0, 5:8, :] = jnp.zeros((3, _KP), jnp.float32)


def _run_nms(cand, candt, B):
    return pl.pallas_call(
        _nms_body,
        grid=(B,),
        in_specs=[
            pl.BlockSpec((1, 16, _KP), lambda b: (b, 0, 0)),
            pl.BlockSpec((1, _KP, 16), lambda b: (b, 0, 0)),
        ],
        out_specs=pl.BlockSpec((1, 8, _KP), lambda b: (b, 0, 0)),
        out_shape=jax.ShapeDtypeStruct((B, 8, _KP), jnp.float32),
        scratch_shapes=[pltpu.VMEM((_KP, _KP), jnp.float32)],
    )(cand, candt)


def kernel(locations, pred_cls, pred_boxes, pred_centerness, image_sizes):
    B, H, W, C = pred_cls.shape
    N = H * W
    K = _TOPN
    cls3 = pred_cls.reshape(B, N, C)
    ctr3 = pred_centerness.reshape(B, N, 1)

    m_loc = _run_locmax(cls3, ctr3, B, N, C)[..., 0]
    _, lidx = lax.top_k(m_loc, _KP)
    lidx = jnp.sort(lidx, axis=1)

    clsg = jnp.take_along_axis(cls3, lidx[..., None], axis=1)
    ctrg = jnp.take_along_axis(ctr3[..., 0], lidx, axis=1)
    pbg = jnp.take_along_axis(
        pred_boxes.reshape(B, N, 4), lidx[..., None], axis=1)
    locg = locations[lidx]

    csg = jax.nn.sigmoid(clsg)
    sg = csg * jax.nn.sigmoid(ctrg)[:, :, None]
    sg = sg * (csg > _CONF).astype(sg.dtype)
    vals, sidx = lax.top_k(sg.reshape(B, _KP * C), K)
    row = sidx // C
    labels = (sidx % C) + 1
    labf = labels.astype(jnp.float32)
    pb = jnp.take_along_axis(pbg, row[..., None], axis=1)
    ploc = jnp.take_along_axis(locg, row[..., None], axis=1)
    hf = image_sizes[0, 0].astype(jnp.float32)
    wf = image_sizes[0, 1].astype(jnp.float32)

    pad = _KP - K
    rows = [ploc[..., 0], ploc[..., 1], pb[..., 0], pb[..., 1], pb[..., 2],
            pb[..., 3], labf]
    rows = [jnp.pad(r, ((0, 0), (0, pad)))[:, None, :] for r in rows]
    rows.append(jnp.broadcast_to(wf, (B, 1, _KP)))
    rows.append(jnp.broadcast_to(hf, (B, 1, _KP)))
    rows.append(jnp.zeros((B, 7, _KP), jnp.float32))
    cand = jnp.concatenate(rows, axis=1)
    candt = jnp.swapaxes(cand, 1, 2)

    out = _run_nms(cand, candt, B)
    det = jnp.swapaxes(out[:, 0:4, :K], 1, 2)
    keep = out[:, 4, :K]

    fsc = vals * keep
    fvals, fidx = lax.top_k(fsc, K)
    fdet = jnp.take_along_axis(det, fidx[..., None], axis=1)
    flab = jnp.take_along_axis(labf, fidx, axis=1)
    flab = flab * (fvals > 0).astype(jnp.float32)
    return jnp.concatenate([fdet, fvals[..., None], flab[..., None]], axis=2)
